# baseline (speedup 1.0000x reference)
import functools

import jax
import jax.numpy as jnp
from jax import lax
from jax.experimental import pallas as pl
from jax.experimental.pallas import tpu as pltpu
from jax.experimental.pallas import tpu_sc as plsc

N_NODES = 10000
NODE_IN = 128
WIDTH = 256
N_EDGES = 320000

NC, NS, L = 2, 16, 16
NW = NC * NS

NB0 = 1000


def _k0_body(x_ref, w1_ref, b1_ref, a_ref, b_ref):
    x = x_ref[...]
    w1a = w1_ref[:NODE_IN, :]
    w1b = w1_ref[NODE_IN:, :]
    a_ref[...] = (
        jnp.dot(x, w1a - w1b, preferred_element_type=jnp.float32) + b1_ref[...]
    )
    b_ref[...] = jnp.dot(x, w1b, preferred_element_type=jnp.float32)


def _precompute_ab(node_feats, W1, b1):
    n = node_feats.shape[0]
    grid = n // NB0
    return pl.pallas_call(
        _k0_body,
        grid=(grid,),
        in_specs=[
            pl.BlockSpec((NB0, NODE_IN), lambda i: (i, 0)),
            pl.BlockSpec((2 * NODE_IN, WIDTH), lambda i: (0, 0)),
            pl.BlockSpec((1, WIDTH), lambda i: (0, 0)),
        ],
        out_specs=[
            pl.BlockSpec((NB0, WIDTH), lambda i: (i, 0)),
            pl.BlockSpec((NB0, WIDTH), lambda i: (i, 0)),
        ],
        out_shape=[
            jax.ShapeDtypeStruct((n, WIDTH), jnp.float32),
            jax.ShapeDtypeStruct((n, WIDTH), jnp.float32),
        ],
    )(node_feats, W1, b1.reshape(1, WIDTH))


K1_CH = 80
K1_EPW = N_EDGES // NW


K1_NCH = K1_EPW // K1_CH


def _k1_body(a_hbm, b_hbm, src_hbm, dst_hbm, g_hbm,
             di0, si0, ar0, br0, di1, si1, ar1, br1, sa0, sb0, sa1, sb1):
    wid = lax.axis_index("s") * NC + lax.axis_index("c")
    base0 = wid * K1_EPW

    def issue(k, di, si, ar, br, sa, sb):
        base = pl.multiple_of(base0 + k * K1_CH, 8)
        pltpu.sync_copy(dst_hbm.at[pl.ds(base, K1_CH)], di)
        pltpu.sync_copy(src_hbm.at[pl.ds(base, K1_CH)], si)
        pltpu.async_copy(a_hbm.at[di], ar, sa)
        pltpu.async_copy(b_hbm.at[si], br, sb)

    def process(k, di, si, ar, br, sa, sb):
        pltpu.make_async_copy(a_hbm.at[di], ar, sa).wait()
        pltpu.make_async_copy(b_hbm.at[si], br, sb).wait()

        def row(r, c):
            for v in range(WIDTH // L):
                sl = pl.ds(v * L, L)
                ar[r, sl] = ar[r, sl] + br[r, sl]
            return c

        lax.fori_loop(0, K1_CH, row, 0)
        base = pl.multiple_of(base0 + k * K1_CH, 8)
        pltpu.sync_copy(ar, g_hbm.at[pl.ds(base, K1_CH)])

    s0 = (di0, si0, ar0, br0, sa0, sb0)
    s1 = (di1, si1, ar1, br1, sa1, sb1)
    issue(0, *s0)

    def pair(q, c):
        k0 = 2 * q
        issue(k0 + 1, *s1)
        process(k0, *s0)
        issue(k0 + 2, *s0)
        process(k0 + 1, *s1)
        return c

    lax.fori_loop(0, (K1_NCH - 1) // 2, pair, 0)
    process(K1_NCH - 1, *s0)


def _gather_add(A, B, src32, dst32):
    mesh = plsc.VectorSubcoreMesh(core_axis_name="c", subcore_axis_name="s")
    f = pl.kernel(
        _k1_body,
        out_type=jax.ShapeDtypeStruct((N_EDGES, WIDTH), jnp.float32),
        mesh=mesh,
        scratch_types=[
            pltpu.VMEM((K1_CH,), jnp.int32),
            pltpu.VMEM((K1_CH,), jnp.int32),
            pltpu.VMEM((K1_CH, WIDTH), jnp.float32),
            pltpu.VMEM((K1_CH, WIDTH), jnp.float32),
            pltpu.VMEM((K1_CH,), jnp.int32),
            pltpu.VMEM((K1_CH,), jnp.int32),
            pltpu.VMEM((K1_CH, WIDTH), jnp.float32),
            pltpu.VMEM((K1_CH, WIDTH), jnp.float32),
            pltpu.SemaphoreType.DMA,
            pltpu.SemaphoreType.DMA,
            pltpu.SemaphoreType.DMA,
            pltpu.SemaphoreType.DMA,
        ],
    )
    return f(A, B, src32, dst32)


EB2 = 512


def _k2_body(g_ref, w2_ref, b2_ref, h_ref):
    h1 = jnp.maximum(g_ref[...], 0.0)
    h2 = jnp.dot(h1, w2_ref[...], preferred_element_type=jnp.float32) + b2_ref[...]
    h_ref[...] = jnp.maximum(h2, 0.0)


def _edge_mlp(G, W2, b2):
    grid = N_EDGES // EB2
    return pl.pallas_call(
        _k2_body,
        grid=(grid,),
        in_specs=[
            pl.BlockSpec((EB2, WIDTH), lambda i: (i, 0)),
            pl.BlockSpec((WIDTH, WIDTH), lambda i: (0, 0)),
            pl.BlockSpec((1, WIDTH), lambda i: (0, 0)),
        ],
        out_specs=pl.BlockSpec((EB2, WIDTH), lambda i: (i, 0)),
        out_shape=jax.ShapeDtypeStruct((N_EDGES, WIDTH), jnp.float32),
    )(G, W2, b2.reshape(1, WIDTH))


BUCKET = 320
MAGIC, MSHIFT = 52429, 24
K3_CH1 = 2000
FLUSH = 2048
K3_CH2 = 64
LIST_CAP = N_EDGES + 2 * FLUSH + K3_CH2


def _lane_gather(x, idx):
    return lax.gather(
        x, idx[:, None],
        dimension_numbers=lax.GatherDimensionNumbers(
            offset_dims=(), collapsed_slice_dims=(0,), start_index_map=(0,)),
        slice_sizes=(1,),
        mode=lax.GatherScatterMode.PROMISE_IN_BOUNDS)


def _k3a_body(dst_hbm, idl_hbm, cnt_hbm, dbuf, lbuf, cbuf):
    wid = lax.axis_index("s") * NC + lax.axis_index("c")
    node_base = wid * BUCKET
    list_base = wid * LIST_CAP
    lanes = lax.iota(jnp.int32, L)
    slotp1 = lanes + 1

    def flush(off, lo_words):
        fo = pl.multiple_of(list_base + off, 8)
        pltpu.sync_copy(lbuf.at[pl.ds(lo_words, FLUSH)],
                        idl_hbm.at[pl.ds(fo, FLUSH)])

    def scan_chunk(k, carry):
        cnt, off = carry
        pltpu.sync_copy(dst_hbm.at[pl.ds(pl.multiple_of(k * K3_CH1, 8), K3_CH1)],
                        dbuf)

        def vreg(i, cnt2):
            d = dbuf[pl.ds(pl.multiple_of(i * L, 8), L)]
            bk = (d * MAGIC) >> MSHIFT
            m = bk == wid
            mi = jnp.where(m, 1, 0)
            ids = (k * K3_CH1 + i * L) + lanes
            packed = (ids << 9) | (d - node_base)
            ps = mi
            for sh in (1, 2, 4, 8):
                g = _lane_gather(ps, jnp.maximum(lanes - sh, 0))
                ps = ps + jnp.where(lanes >= sh, g, 0)
            tot = ps[L - 1]

            def slow(c2):
                pos = jnp.zeros((L,), jnp.int32)
                for st in (8, 4, 2, 1):
                    ps_at = _lane_gather(ps, pos + (st - 1))
                    pos = pos + jnp.where(ps_at < slotp1, st, 0)
                compact = _lane_gather(packed, jnp.minimum(pos, L - 1))
                lbuf[pl.ds(c2, L)] = compact
                return c2 + tot

            return lax.cond(tot > 0, slow, lambda c2: c2, cnt2)

        cnt = lax.fori_loop(0, K3_CH1 // L, vreg, cnt)

        def do_flush(args):
            cnt, off = args
            flush(off, 0)

            def mv(i, c):
                sl = pl.ds(i * L, L)
                lbuf[sl] = lbuf.at[pl.ds(FLUSH, FLUSH + L)][sl]
                return c

            lax.fori_loop(0, FLUSH // L + 1, mv, 0)
            return cnt - FLUSH, off + FLUSH

        return lax.cond(cnt >= FLUSH, do_flush, lambda a: a, (cnt, off))

    cnt, off = lax.fori_loop(0, N_EDGES // K3_CH1, scan_chunk,
                             (jnp.int32(0), jnp.int32(0)))
    flush(cnt * 0 + off, 0)
    flush(cnt * 0 + off + FLUSH, FLUSH)
    total = off + cnt
    cbuf[pl.ds(0, L)] = jnp.zeros((L,), jnp.int32) + total
    pltpu.sync_copy(cbuf, cnt_hbm.at[pl.ds(pl.multiple_of(wid * L, 8), L)])


def _route(dst32):
    mesh = plsc.VectorSubcoreMesh(core_axis_name="c", subcore_axis_name="s")
    f = pl.kernel(
        _k3a_body,
        out_type=[
            jax.ShapeDtypeStruct((NW * LIST_CAP,), jnp.int32),
            jax.ShapeDtypeStruct((NW * L,), jnp.int32),
        ],
        mesh=mesh,
        scratch_types=[
            pltpu.VMEM((K3_CH1,), jnp.int32),
            pltpu.VMEM((2 * FLUSH + 2 * L,), jnp.int32),
            pltpu.VMEM((L,), jnp.int32),
        ],
    )
    return f(dst32)


def _k3b_body(h_hbm, idl_hbm, cnt_hbm, out_hbm,
              vbuf0, vbuf1, idbuf0, idbuf1, hbuf0, hbuf1, table, cbuf,
              sem0, sem1):
    wid = lax.axis_index("s") * NC + lax.axis_index("c")
    node_base = wid * BUCKET
    list_base = wid * LIST_CAP

    pltpu.sync_copy(cnt_hbm.at[pl.ds(pl.multiple_of(wid * L, 8), L)], cbuf)
    total = cbuf[pl.ds(0, L)][0]

    zeros = jnp.zeros((L,), jnp.float32)

    def zrow(r, c):
        for v in range(WIDTH // L):
            table[r, pl.ds(v * L, L)] = zeros
        return c

    lax.fori_loop(0, BUCKET, zrow, 0)

    nch = (total + K3_CH2 - 1) // K3_CH2

    def issue(k2, vbuf, idbuf, hbuf, sem):
        lo = pl.multiple_of(list_base + k2 * K3_CH2, 8)
        pltpu.sync_copy(idl_hbm.at[pl.ds(lo, K3_CH2)],
                        vbuf.at[pl.ds(0, K3_CH2)])
        for v in range(K3_CH2 // L):
            sl = pl.ds(v * L, L)
            gid = vbuf[sl] >> 9
            idbuf[sl] = jnp.minimum(jnp.maximum(gid, 0), N_EDGES - 1)
        pltpu.async_copy(h_hbm.at[idbuf], hbuf, sem)

    def process(k2, vbuf, idbuf, hbuf, sem):
        pltpu.make_async_copy(h_hbm.at[idbuf], hbuf, sem).wait()
        take = jnp.minimum(K3_CH2, total - k2 * K3_CH2)

        def edge(j, cc):
            row = vbuf[pl.ds(j, L)][0] & 511
            for v in range(WIDTH // L):
                sl = pl.ds(v * L, L)
                table[row, sl] = jnp.maximum(table[row, sl], hbuf[j, sl])
            return cc

        lax.fori_loop(0, take, edge, 0)

    @pl.when(nch > 0)
    def _():
        issue(0, vbuf0, idbuf0, hbuf0, sem0)

    def pair(q, c):
        k0 = 2 * q
        k1 = k0 + 1

        @pl.when(k1 < nch)
        def _():
            issue(k1, vbuf1, idbuf1, hbuf1, sem1)

        process(k0, vbuf0, idbuf0, hbuf0, sem0)

        @pl.when(k1 < nch)
        def _():
            @pl.when(k1 + 1 < nch)
            def _():
                issue(k1 + 1, vbuf0, idbuf0, hbuf0, sem0)

            process(k1, vbuf1, idbuf1, hbuf1, sem1)

        return c

    lax.fori_loop(0, (nch + 1) // 2, pair, 0)
    pltpu.sync_copy(table, out_hbm.at[pl.ds(pl.multiple_of(node_base, 8), BUCKET)])


def _scatter_max(H, idl, cnts):
    mesh = plsc.VectorSubcoreMesh(core_axis_name="c", subcore_axis_name="s")
    f = pl.kernel(
        _k3b_body,
        out_type=jax.ShapeDtypeStruct((NW * BUCKET, WIDTH), jnp.float32),
        mesh=mesh,
        scratch_types=[
            pltpu.VMEM((K3_CH2 + L,), jnp.int32),
            pltpu.VMEM((K3_CH2 + L,), jnp.int32),
            pltpu.VMEM((K3_CH2,), jnp.int32),
            pltpu.VMEM((K3_CH2,), jnp.int32),
            pltpu.VMEM((K3_CH2, WIDTH), jnp.float32),
            pltpu.VMEM((K3_CH2, WIDTH), jnp.float32),
            pltpu.VMEM((BUCKET, WIDTH), jnp.float32),
            pltpu.VMEM((L,), jnp.int32),
            pltpu.SemaphoreType.DMA,
            pltpu.SemaphoreType.DMA,
        ],
    )
    out_pad = f(H, idl, cnts)
    return out_pad[:N_NODES]


def kernel(node_feats, edge_index, W1, b1, W2, b2):
    src32 = edge_index[0].astype(jnp.int32)
    dst32 = edge_index[1].astype(jnp.int32)
    idl, cnts = _route(dst32)
    A, B = _precompute_ab(node_feats, W1, b1)
    G = _gather_add(A, B, src32, dst32)
    H = _edge_mlp(G, W2, b2)
    return _scatter_max(H, idl, cnts)


# speedup vs baseline: 1.0865x; 1.0865x over previous
import functools

import jax
import jax.numpy as jnp
from jax import lax
from jax.experimental import pallas as pl
from jax.experimental.pallas import tpu as pltpu
from jax.experimental.pallas import tpu_sc as plsc

N_NODES = 10000
NODE_IN = 128
WIDTH = 256
N_EDGES = 320000

NC, NS, L = 2, 16, 16
NW = NC * NS

NB0 = 1000


def _k0_body(x_ref, w1_ref, b1_ref, a_ref, b_ref):
    x = x_ref[...]
    w1a = w1_ref[:NODE_IN, :]
    w1b = w1_ref[NODE_IN:, :]
    a_ref[...] = (
        jnp.dot(x, w1a - w1b, preferred_element_type=jnp.float32) + b1_ref[...]
    )
    b_ref[...] = jnp.dot(x, w1b, preferred_element_type=jnp.float32)


def _precompute_ab(node_feats, W1, b1):
    n = node_feats.shape[0]
    grid = n // NB0
    return pl.pallas_call(
        _k0_body,
        grid=(grid,),
        in_specs=[
            pl.BlockSpec((NB0, NODE_IN), lambda i: (i, 0)),
            pl.BlockSpec((2 * NODE_IN, WIDTH), lambda i: (0, 0)),
            pl.BlockSpec((1, WIDTH), lambda i: (0, 0)),
        ],
        out_specs=[
            pl.BlockSpec((NB0, WIDTH), lambda i: (i, 0)),
            pl.BlockSpec((NB0, WIDTH), lambda i: (i, 0)),
        ],
        out_shape=[
            jax.ShapeDtypeStruct((n, WIDTH), jnp.float32),
            jax.ShapeDtypeStruct((n, WIDTH), jnp.float32),
        ],
    )(node_feats, W1, b1.reshape(1, WIDTH))


K1_CH = 80
K1_EPW = N_EDGES // NW


K1_NCH = K1_EPW // K1_CH


def _k1_body(a_hbm, b_hbm, src_hbm, dst_hbm, g_hbm,
             di0, si0, ar0, br0, di1, si1, ar1, br1, sa0, sb0, sa1, sb1):
    wid = lax.axis_index("s") * NC + lax.axis_index("c")
    base0 = wid * K1_EPW

    def issue(k, di, si, ar, br, sa, sb):
        base = pl.multiple_of(base0 + k * K1_CH, 8)
        pltpu.sync_copy(dst_hbm.at[pl.ds(base, K1_CH)], di)
        pltpu.sync_copy(src_hbm.at[pl.ds(base, K1_CH)], si)
        pltpu.async_copy(a_hbm.at[di], ar, sa)
        pltpu.async_copy(b_hbm.at[si], br, sb)

    def process(k, di, si, ar, br, sa, sb):
        pltpu.make_async_copy(a_hbm.at[di], ar, sa).wait()
        pltpu.make_async_copy(b_hbm.at[si], br, sb).wait()

        def row(r, c):
            for v in range(WIDTH // L):
                sl = pl.ds(v * L, L)
                ar[r, sl] = ar[r, sl] + br[r, sl]
            return c

        lax.fori_loop(0, K1_CH, row, 0)
        base = pl.multiple_of(base0 + k * K1_CH, 8)
        pltpu.sync_copy(ar, g_hbm.at[pl.ds(base, K1_CH)])

    s0 = (di0, si0, ar0, br0, sa0, sb0)
    s1 = (di1, si1, ar1, br1, sa1, sb1)
    issue(0, *s0)

    def pair(q, c):
        k0 = 2 * q
        issue(k0 + 1, *s1)
        process(k0, *s0)
        issue(k0 + 2, *s0)
        process(k0 + 1, *s1)
        return c

    lax.fori_loop(0, (K1_NCH - 1) // 2, pair, 0)
    process(K1_NCH - 1, *s0)


def _gather_add(A, B, src32, dst32):
    mesh = plsc.VectorSubcoreMesh(core_axis_name="c", subcore_axis_name="s")
    f = pl.kernel(
        _k1_body,
        out_type=jax.ShapeDtypeStruct((N_EDGES, WIDTH), jnp.float32),
        mesh=mesh,
        scratch_types=[
            pltpu.VMEM((K1_CH,), jnp.int32),
            pltpu.VMEM((K1_CH,), jnp.int32),
            pltpu.VMEM((K1_CH, WIDTH), jnp.float32),
            pltpu.VMEM((K1_CH, WIDTH), jnp.float32),
            pltpu.VMEM((K1_CH,), jnp.int32),
            pltpu.VMEM((K1_CH,), jnp.int32),
            pltpu.VMEM((K1_CH, WIDTH), jnp.float32),
            pltpu.VMEM((K1_CH, WIDTH), jnp.float32),
            pltpu.SemaphoreType.DMA,
            pltpu.SemaphoreType.DMA,
            pltpu.SemaphoreType.DMA,
            pltpu.SemaphoreType.DMA,
        ],
    )
    return f(A, B, src32, dst32)


EB2 = 512


def _k2_body(g_ref, w2_ref, b2_ref, h_ref):
    h1 = jnp.maximum(g_ref[...], 0.0)
    h2 = jnp.dot(h1, w2_ref[...], preferred_element_type=jnp.float32) + b2_ref[...]
    h_ref[...] = jnp.maximum(h2, 0.0)


def _edge_mlp(G, W2, b2):
    grid = N_EDGES // EB2
    return pl.pallas_call(
        _k2_body,
        grid=(grid,),
        in_specs=[
            pl.BlockSpec((EB2, WIDTH), lambda i: (i, 0)),
            pl.BlockSpec((WIDTH, WIDTH), lambda i: (0, 0)),
            pl.BlockSpec((1, WIDTH), lambda i: (0, 0)),
        ],
        out_specs=pl.BlockSpec((EB2, WIDTH), lambda i: (i, 0)),
        out_shape=jax.ShapeDtypeStruct((N_EDGES, WIDTH), jnp.float32),
    )(G, W2, b2.reshape(1, WIDTH))


BUCKET = 320
MAGIC, MSHIFT = 52429, 24
K3_CH1 = 2000
FLUSH = 2048
K3_CH2 = 64
LIST_CAP = N_EDGES + 2 * FLUSH + K3_CH2


def _lane_gather(x, idx):
    return lax.gather(
        x, idx[:, None],
        dimension_numbers=lax.GatherDimensionNumbers(
            offset_dims=(), collapsed_slice_dims=(0,), start_index_map=(0,)),
        slice_sizes=(1,),
        mode=lax.GatherScatterMode.PROMISE_IN_BOUNDS)


def _k3a_body(dst_hbm, idl_hbm, cnt_hbm, dbuf, lbuf, cbuf):
    wid = lax.axis_index("s") * NC + lax.axis_index("c")
    node_base = wid * BUCKET
    list_base = wid * LIST_CAP
    lanes = lax.iota(jnp.int32, L)
    slotp1 = lanes + 1

    def flush(off, lo_words):
        fo = pl.multiple_of(list_base + off, 8)
        pltpu.sync_copy(lbuf.at[pl.ds(lo_words, FLUSH)],
                        idl_hbm.at[pl.ds(fo, FLUSH)])

    def scan_chunk(k, carry):
        cnt, off = carry
        pltpu.sync_copy(dst_hbm.at[pl.ds(pl.multiple_of(k * K3_CH1, 8), K3_CH1)],
                        dbuf)

        def vreg(i, cnt2):
            d = dbuf[pl.ds(pl.multiple_of(i * L, 8), L)]
            bk = (d * MAGIC) >> MSHIFT
            m = bk == wid
            mi = jnp.where(m, 1, 0)
            ids = (k * K3_CH1 + i * L) + lanes
            packed = (ids << 9) | (d - node_base)
            ps = mi
            for sh in (1, 2, 4, 8):
                g = _lane_gather(ps, jnp.maximum(lanes - sh, 0))
                ps = ps + jnp.where(lanes >= sh, g, 0)
            pos = jnp.zeros((L,), jnp.int32)
            for st in (8, 4, 2, 1):
                ps_at = _lane_gather(ps, pos + (st - 1))
                pos = pos + jnp.where(ps_at < slotp1, st, 0)
            compact = _lane_gather(packed, jnp.minimum(pos, L - 1))
            lbuf[pl.ds(cnt2, L)] = compact
            return cnt2 + ps[L - 1]

        cnt = lax.fori_loop(0, K3_CH1 // L, vreg, cnt)

        def do_flush(args):
            cnt, off = args
            flush(off, 0)

            def mv(i, c):
                sl = pl.ds(i * L, L)
                lbuf[sl] = lbuf.at[pl.ds(FLUSH, FLUSH + L)][sl]
                return c

            lax.fori_loop(0, FLUSH // L + 1, mv, 0)
            return cnt - FLUSH, off + FLUSH

        return lax.cond(cnt >= FLUSH, do_flush, lambda a: a, (cnt, off))

    cnt, off = lax.fori_loop(0, N_EDGES // K3_CH1, scan_chunk,
                             (jnp.int32(0), jnp.int32(0)))
    flush(cnt * 0 + off, 0)
    flush(cnt * 0 + off + FLUSH, FLUSH)
    total = off + cnt
    cbuf[pl.ds(0, L)] = jnp.zeros((L,), jnp.int32) + total
    pltpu.sync_copy(cbuf, cnt_hbm.at[pl.ds(pl.multiple_of(wid * L, 8), L)])


def _route(dst32):
    mesh = plsc.VectorSubcoreMesh(core_axis_name="c", subcore_axis_name="s")
    f = pl.kernel(
        _k3a_body,
        out_type=[
            jax.ShapeDtypeStruct((NW * LIST_CAP,), jnp.int32),
            jax.ShapeDtypeStruct((NW * L,), jnp.int32),
        ],
        mesh=mesh,
        scratch_types=[
            pltpu.VMEM((K3_CH1,), jnp.int32),
            pltpu.VMEM((2 * FLUSH + 2 * L,), jnp.int32),
            pltpu.VMEM((L,), jnp.int32),
        ],
    )
    return f(dst32)


def _k3b_body(h_hbm, idl_hbm, cnt_hbm, out_hbm,
              vbuf0, vbuf1, idbuf0, idbuf1, hbuf0, hbuf1, table, cbuf,
              sem0, sem1):
    wid = lax.axis_index("s") * NC + lax.axis_index("c")
    node_base = wid * BUCKET
    list_base = wid * LIST_CAP

    pltpu.sync_copy(cnt_hbm.at[pl.ds(pl.multiple_of(wid * L, 8), L)], cbuf)
    total = cbuf[pl.ds(0, L)][0]

    zeros = jnp.zeros((L,), jnp.float32)

    def zrow(r, c):
        for v in range(WIDTH // L):
            table[r, pl.ds(v * L, L)] = zeros
        return c

    lax.fori_loop(0, BUCKET, zrow, 0)

    nch = (total + K3_CH2 - 1) // K3_CH2

    def issue(k2, vbuf, idbuf, hbuf, sem):
        lo = pl.multiple_of(list_base + k2 * K3_CH2, 8)
        pltpu.sync_copy(idl_hbm.at[pl.ds(lo, K3_CH2)],
                        vbuf.at[pl.ds(0, K3_CH2)])
        for v in range(K3_CH2 // L):
            sl = pl.ds(v * L, L)
            gid = vbuf[sl] >> 9
            idbuf[sl] = jnp.minimum(jnp.maximum(gid, 0), N_EDGES - 1)
        pltpu.async_copy(h_hbm.at[idbuf], hbuf, sem)

    def process(k2, vbuf, idbuf, hbuf, sem):
        pltpu.make_async_copy(h_hbm.at[idbuf], hbuf, sem).wait()
        take = jnp.minimum(K3_CH2, total - k2 * K3_CH2)

        def edge(j, cc):
            row = vbuf[pl.ds(j, L)][0] & 511
            for v in range(WIDTH // L):
                sl = pl.ds(v * L, L)
                table[row, sl] = jnp.maximum(table[row, sl], hbuf[j, sl])
            return cc

        lax.fori_loop(0, take, edge, 0)

    @pl.when(nch > 0)
    def _():
        issue(0, vbuf0, idbuf0, hbuf0, sem0)

    def pair(q, c):
        k0 = 2 * q
        k1 = k0 + 1

        @pl.when(k1 < nch)
        def _():
            issue(k1, vbuf1, idbuf1, hbuf1, sem1)

        process(k0, vbuf0, idbuf0, hbuf0, sem0)

        @pl.when(k1 < nch)
        def _():
            @pl.when(k1 + 1 < nch)
            def _():
                issue(k1 + 1, vbuf0, idbuf0, hbuf0, sem0)

            process(k1, vbuf1, idbuf1, hbuf1, sem1)

        return c

    lax.fori_loop(0, (nch + 1) // 2, pair, 0)
    pltpu.sync_copy(table, out_hbm.at[pl.ds(pl.multiple_of(node_base, 8), BUCKET)])


def _scatter_max(H, idl, cnts):
    mesh = plsc.VectorSubcoreMesh(core_axis_name="c", subcore_axis_name="s")
    f = pl.kernel(
        _k3b_body,
        out_type=jax.ShapeDtypeStruct((NW * BUCKET, WIDTH), jnp.float32),
        mesh=mesh,
        scratch_types=[
            pltpu.VMEM((K3_CH2 + L,), jnp.int32),
            pltpu.VMEM((K3_CH2 + L,), jnp.int32),
            pltpu.VMEM((K3_CH2,), jnp.int32),
            pltpu.VMEM((K3_CH2,), jnp.int32),
            pltpu.VMEM((K3_CH2, WIDTH), jnp.float32),
            pltpu.VMEM((K3_CH2, WIDTH), jnp.float32),
            pltpu.VMEM((BUCKET, WIDTH), jnp.float32),
            pltpu.VMEM((L,), jnp.int32),
            pltpu.SemaphoreType.DMA,
            pltpu.SemaphoreType.DMA,
        ],
    )
    out_pad = f(H, idl, cnts)
    return out_pad[:N_NODES]


def kernel(node_feats, edge_index, W1, b1, W2, b2):
    src32 = edge_index[0].astype(jnp.int32)
    dst32 = edge_index[1].astype(jnp.int32)
    idl, cnts = _route(dst32)
    A, B = _precompute_ab(node_feats, W1, b1)
    G = _gather_add(A, B, src32, dst32)
    H = _edge_mlp(G, W2, b2)
    return _scatter_max(H, idl, cnts)
